# Initial kernel scaffold; baseline (speedup 1.0000x reference)
import jax
import jax.numpy as jnp
from jax import lax
from jax.experimental import pallas as pl
from jax.experimental.pallas import tpu as pltpu
from jax.experimental.pallas import tpu_sc as plsc

BATCH = 16384
N_FIELDS = 100
NC = 2
NS = 16
NW = NC * NS
BPW = BATCH // NW
EPW = BPW * N_FIELDS
LANES = 16


def _wide_body(idx_hbm, table_hbm, bias_hbm, out_hbm,
               idx_v, vals_v, out_v, bias_v, sem):
    wid = lax.axis_index("s") * NC + lax.axis_index("c")
    ebase = wid * EPW
    pltpu.sync_copy(idx_hbm.at[pl.ds(ebase, EPW)], idx_v)
    pltpu.sync_copy(bias_hbm, bias_v)
    pltpu.async_copy(table_hbm.at[idx_v], vals_v, sem).wait()

    lanes = lax.iota(jnp.int32, (LANES,))

    def g_body(g, carry):
        row0 = g * (LANES * N_FIELDS)

        def f_body(f, acc):
            gi = row0 + lanes * N_FIELDS + f
            return acc + plsc.load_gather(vals_v, [gi])

        acc = lax.fori_loop(0, N_FIELDS, f_body, bias_v[...])
        out_v[pl.ds(g * LANES, LANES)] = acc
        return carry

    lax.fori_loop(0, BPW // LANES, g_body, 0)
    pltpu.sync_copy(out_v, out_hbm.at[pl.ds(wid * BPW, BPW)])


def kernel(inputs, table, bias):
    idx_flat = inputs.reshape(-1).astype(jnp.int32)
    table_flat = table.reshape(-1)
    bias16 = jnp.broadcast_to(bias.astype(jnp.float32), (LANES,))
    mesh = plsc.VectorSubcoreMesh(core_axis_name="c", subcore_axis_name="s")
    out = pl.kernel(
        _wide_body,
        out_type=jax.ShapeDtypeStruct((BATCH,), jnp.float32),
        mesh=mesh,
        scratch_types=[
            pltpu.VMEM((EPW,), jnp.int32),
            pltpu.VMEM((EPW,), jnp.float32),
            pltpu.VMEM((BPW,), jnp.float32),
            pltpu.VMEM((LANES,), jnp.float32),
            pltpu.SemaphoreType.DMA,
        ],
    )(idx_flat, table_flat, bias16)
    return out.reshape(BATCH, 1)


# speedup vs baseline: 1.1756x; 1.1756x over previous
import jax
import jax.numpy as jnp
from jax import lax
from jax.experimental import pallas as pl
from jax.experimental.pallas import tpu as pltpu
from jax.experimental.pallas import tpu_sc as plsc

BATCH = 16384
N_FIELDS = 100
NC = 2
NS = 16
NW = NC * NS
BPW = BATCH // NW
EPW = BPW * N_FIELDS
LANES = 16


def _wide_body(idx_hbm, table_hbm, bias_hbm, out_hbm,
               idx_v, vals_v, out_v, bias_v, sem):
    wid = lax.axis_index("s") * NC + lax.axis_index("c")
    ebase = wid * EPW
    pltpu.sync_copy(idx_hbm.at[pl.ds(ebase, EPW)], idx_v)
    pltpu.sync_copy(bias_hbm, bias_v)
    pltpu.async_copy(table_hbm.at[idx_v], vals_v, sem).wait()

    def g_body(g, carry):
        col0 = g * LANES

        def f_body(f, acc):
            return acc + vals_v[pl.ds(f * BPW + col0, LANES)]

        acc = lax.fori_loop(0, N_FIELDS, f_body, bias_v[...])
        out_v[pl.ds(col0, LANES)] = acc
        return carry

    lax.fori_loop(0, BPW // LANES, g_body, 0)
    pltpu.sync_copy(out_v, out_hbm.at[pl.ds(wid * BPW, BPW)])


def kernel(inputs, table, bias):
    idx_arranged = (
        inputs.astype(jnp.int32)
        .reshape(NW, BPW, N_FIELDS)
        .transpose(0, 2, 1)
        .reshape(-1)
    )
    table_flat = table.reshape(-1)
    bias16 = jnp.broadcast_to(bias.astype(jnp.float32), (LANES,))
    mesh = plsc.VectorSubcoreMesh(core_axis_name="c", subcore_axis_name="s")
    out = pl.kernel(
        _wide_body,
        out_type=jax.ShapeDtypeStruct((BATCH,), jnp.float32),
        mesh=mesh,
        scratch_types=[
            pltpu.VMEM((EPW,), jnp.int32),
            pltpu.VMEM((EPW,), jnp.float32),
            pltpu.VMEM((BPW,), jnp.float32),
            pltpu.VMEM((LANES,), jnp.float32),
            pltpu.SemaphoreType.DMA,
        ],
    )(idx_arranged, table_flat, bias16)
    return out.reshape(BATCH, 1)


# speedup vs baseline: 1.8646x; 1.5861x over previous
import jax
import jax.numpy as jnp
from jax import lax
from jax.experimental import pallas as pl
from jax.experimental.pallas import tpu as pltpu
from jax.experimental.pallas import tpu_sc as plsc

BATCH = 16384
N_FIELDS = 100
WIDE = 1000001
NC = 2
NS = 16
NW = NC * NS
BPW = BATCH // NW
EPW = BPW * N_FIELDS
LANES = 16


def _wide_body(idx_hbm, table_hbm, bias_hbm, out_hbm,
               idx_v, vals_v, out_v, bias_v, sem, gsem):
    wid = lax.axis_index("s") * NC + lax.axis_index("c")
    b0 = wid * BPW

    def fire(fi, carry):
        pltpu.async_copy(idx_hbm.at[pl.ds(fi * BATCH + b0, BPW)],
                         idx_v.at[pl.ds(fi * BPW, BPW)], sem)
        return carry

    lax.fori_loop(0, N_FIELDS, fire, 0)
    pltpu.sync_copy(bias_hbm, bias_v)

    def drain(fi, carry):
        pltpu.make_async_copy(idx_hbm.at[pl.ds(0, BPW)],
                              idx_v.at[pl.ds(0, BPW)], sem).wait()
        return carry

    lax.fori_loop(0, N_FIELDS, drain, 0)

    tview = table_hbm.at[0]
    pltpu.async_copy(tview.at[idx_v], vals_v, gsem).wait()

    def g_body(g, carry):
        col0 = g * LANES

        def f_body(f, acc):
            return acc + vals_v[pl.ds(f * BPW + col0, LANES)]

        acc = lax.fori_loop(0, N_FIELDS, f_body, bias_v[...])
        out_v[pl.ds(col0, LANES)] = acc
        return carry

    lax.fori_loop(0, BPW // LANES, g_body, 0)
    pltpu.sync_copy(out_v, out_hbm.at[pl.ds(b0, BPW)])


def kernel(inputs, table, bias):
    idx_t = inputs.astype(jnp.int32).T.reshape(-1)
    table2 = table.reshape(1, WIDE)
    bias16 = jnp.broadcast_to(bias.astype(jnp.float32), (LANES,))
    mesh = plsc.VectorSubcoreMesh(core_axis_name="c", subcore_axis_name="s")
    out = pl.kernel(
        _wide_body,
        out_type=jax.ShapeDtypeStruct((BATCH,), jnp.float32),
        mesh=mesh,
        scratch_types=[
            pltpu.VMEM((EPW,), jnp.int32),
            pltpu.VMEM((EPW,), jnp.float32),
            pltpu.VMEM((BPW,), jnp.float32),
            pltpu.VMEM((LANES,), jnp.float32),
            pltpu.SemaphoreType.DMA,
            pltpu.SemaphoreType.DMA,
        ],
    )(idx_t, table2, bias16)
    return out.reshape(BATCH, 1)


# speedup vs baseline: 2.0297x; 1.0886x over previous
import jax
import jax.numpy as jnp
from jax import lax
from jax.experimental import pallas as pl
from jax.experimental.pallas import tpu as pltpu
from jax.experimental.pallas import tpu_sc as plsc

BATCH = 16384
N_FIELDS = 100
WIDE = 1000001
NC = 2
NS = 16
NW = NC * NS
BPW = BATCH // NW
EPW = BPW * N_FIELDS
LANES = 16
NCHUNK = 4
CPF = N_FIELDS // NCHUNK
CEL = CPF * BPW


def _tree_sum(terms):
    while len(terms) > 1:
        nxt = [terms[i] + terms[i + 1] for i in range(0, len(terms) - 1, 2)]
        if len(terms) % 2:
            nxt.append(terms[-1])
        terms = nxt
    return terms[0]


def _wide_body(idx_hbm, table_hbm, bias_hbm, out_hbm,
               idx_v, vals_v, out_v, bias_v, sem, gsem0, gsem1):
    wid = lax.axis_index("s") * NC + lax.axis_index("c")
    b0 = wid * BPW

    def fire(fi, carry):
        pltpu.async_copy(idx_hbm.at[pl.ds(fi * BATCH + b0, BPW)],
                         idx_v.at[pl.ds(fi * BPW, BPW)], sem)
        return carry

    lax.fori_loop(0, N_FIELDS, fire, 0)
    pltpu.sync_copy(bias_hbm, bias_v)

    def drain(fi, carry):
        pltpu.make_async_copy(idx_hbm.at[pl.ds(0, BPW)],
                              idx_v.at[pl.ds(0, BPW)], sem).wait()
        return carry

    lax.fori_loop(0, N_FIELDS, drain, 0)

    tview = table_hbm.at[0]
    gsems = (gsem0, gsem1)

    def fire_gather(c):
        return pltpu.async_copy(
            tview.at[idx_v.at[pl.ds(c * CEL, CEL)]],
            vals_v.at[pl.ds(c * CEL, CEL)], gsems[c % 2])

    def reduce_chunk(c):
        base = c * CEL

        def g_body(g, carry):
            col0 = g * LANES
            terms = [vals_v[pl.ds(base + f * BPW + col0, LANES)]
                     for f in range(CPF)]
            if c == 0:
                terms.append(bias_v[...])
            else:
                terms.append(out_v[pl.ds(col0, LANES)])
            out_v[pl.ds(col0, LANES)] = _tree_sum(terms)
            return carry

        lax.fori_loop(0, BPW // LANES, g_body, 0)

    handles = [fire_gather(0), fire_gather(1)]
    for c in range(NCHUNK):
        handles[c].wait()
        if c + 2 < NCHUNK:
            handles.append(fire_gather(c + 2))
        reduce_chunk(c)

    pltpu.sync_copy(out_v, out_hbm.at[pl.ds(b0, BPW)])


def kernel(inputs, table, bias):
    idx_t = inputs.astype(jnp.int32).T.reshape(-1)
    table2 = table.reshape(1, WIDE)
    bias16 = jnp.broadcast_to(bias.astype(jnp.float32), (LANES,))
    mesh = plsc.VectorSubcoreMesh(core_axis_name="c", subcore_axis_name="s")
    out = pl.kernel(
        _wide_body,
        out_type=jax.ShapeDtypeStruct((BATCH,), jnp.float32),
        mesh=mesh,
        scratch_types=[
            pltpu.VMEM((EPW,), jnp.int32),
            pltpu.VMEM((EPW,), jnp.float32),
            pltpu.VMEM((BPW,), jnp.float32),
            pltpu.VMEM((LANES,), jnp.float32),
            pltpu.SemaphoreType.DMA,
            pltpu.SemaphoreType.DMA,
            pltpu.SemaphoreType.DMA,
        ],
    )(idx_t, table2, bias16)
    return out.reshape(BATCH, 1)
